# Initial kernel scaffold; baseline (speedup 1.0000x reference)
import jax
import jax.numpy as jnp
from jax.experimental import pallas as pl


def kernel(x, edge_index, edge_weight, W1, b1, W2, b2, Wo, bo):
    raise NotImplementedError("write your pallas kernel here")


# speedup vs baseline: 2.3691x; 2.3691x over previous
import functools

import jax
import jax.numpy as jnp
from jax import lax
from jax.experimental import pallas as pl
from jax.experimental.pallas import tpu as pltpu
from jax.experimental.pallas import tpu_sc as plsc

N = 10000
F = 128
E = 320000
C = 40
ALPHA = 1.0
N_POWERS = 10
C1 = ALPHA / (1.0 + ALPHA)
C2 = 1.0 / (1.0 + ALPHA)

NS = 16
NC = 2
NW = NC * NS
EB = 128
NPAD = 10240
ROWS = NPAD // NW
EPADL = E + 2 * EB


def _pad_rows(a):
    return jnp.concatenate(
        [a, jnp.zeros((NPAD - N, a.shape[1]), jnp.float32)], axis=0
    )


def _prep_body(x_ref, w_ref, xc_ref, sxc_ref, ws_ref):
    x = x_ref[...]
    m = jnp.mean(x, axis=0, keepdims=True)
    xc = x - m
    xc_ref[...] = _pad_rows(xc)
    sxc_ref[...] = _pad_rows(C2 * xc)
    ws_ref[...] = C1 * w_ref[...]


def _prep_call(x, w_sorted):
    return pl.pallas_call(
        _prep_body,
        out_shape=[
            jax.ShapeDtypeStruct((NPAD, F), jnp.float32),
            jax.ShapeDtypeStruct((NPAD, F), jnp.float32),
            jax.ShapeDtypeStruct(w_sorted.shape, jnp.float32),
        ],
    )(x, w_sorted)


def _mid_body(inv_ref, W_ref, b_ref, xc_ref, sxc_ref):
    h = jnp.dot(inv_ref[:N, :], W_ref[...], preferred_element_type=jnp.float32)
    h = h + b_ref[...]
    m = jnp.mean(h, axis=0, keepdims=True)
    xc = h - m
    xc_ref[...] = _pad_rows(xc)
    sxc_ref[...] = _pad_rows(C2 * xc)


def _mid_call(inv, W, b):
    return pl.pallas_call(
        _mid_body,
        out_shape=[
            jax.ShapeDtypeStruct((NPAD, F), jnp.float32),
            jax.ShapeDtypeStruct((NPAD, F), jnp.float32),
        ],
    )(inv, W, b)


def _out_body(inv_ref, W2_ref, b2_ref, Wo_ref, bo_ref, o_ref):
    h = jnp.dot(inv_ref[:N, :], W2_ref[...], preferred_element_type=jnp.float32)
    h = h + b2_ref[...]
    o_ref[...] = jnp.dot(h, Wo_ref[...], preferred_element_type=jnp.float32) + bo_ref[...]


def _out_call(inv, W2, b2, Wo, bo):
    return pl.pallas_call(
        _out_body,
        out_shape=jax.ShapeDtypeStruct((N, C), jnp.float32),
    )(inv, W2, b2, Wo, bo)


_BC_F = lax.GatherDimensionNumbers(
    offset_dims=(), collapsed_slice_dims=(0,), start_index_map=(0,)
)


def _bcast_lane(v16, j):
    idx = jnp.full((16, 1), j, dtype=jnp.int32)
    return lax.gather(
        v16, idx, _BC_F, (1,), mode=lax.GatherScatterMode.PROMISE_IN_BOUNDS
    )


def _step_body(h_in, sxc, srcb, dstb, wb, bnd, h_out,
               y1d, gbuf, src_v, dst_v, w_v, bvm):
    cid = lax.axis_index("c")
    sid = lax.axis_index("s")
    wid = cid * NS + sid
    f0 = pl.multiple_of(wid * (ROWS * F), ROWS * F)

    pltpu.sync_copy(sxc.at[pl.ds(f0, ROWS * F)], y1d)

    pltpu.sync_copy(bnd.at[cid, sid], bvm)
    s_ex = bvm[0]
    e_ex = bvm[1]
    b0 = bvm[2][0]
    nb = bvm[3][0]

    iota = lax.iota(jnp.int32, 16)
    rbase = wid * ROWS

    def blk(b, carry):
        e0 = pl.multiple_of(b0 + b * EB, EB)
        pltpu.sync_copy(srcb.at[pl.ds(e0, EB)], src_v)
        pltpu.sync_copy(dstb.at[pl.ds(e0, EB)], dst_v)
        pltpu.sync_copy(wb.at[pl.ds(e0, EB)], w_v)
        pltpu.sync_copy(h_in.at[src_v], gbuf)

        def grp(g, c2):
            gid = jnp.full((16,), e0 + g * 16, jnp.int32) + iota
            valid = (gid >= s_ex) & (gid < e_ex)
            w16 = jnp.where(valid, w_v[pl.ds(g * 16, 16)], 0.0)
            d16 = jnp.clip(dst_v[pl.ds(g * 16, 16)] - rbase, 0, ROWS - 1)
            for j in range(16):
                bw = _bcast_lane(w16, j)
                bd = _bcast_lane(d16, j) * F
                e = g * 16 + j
                for k in range(F // 16):
                    lin = bd + (iota + k * 16)
                    val = gbuf[e, pl.ds(k * 16, 16)] * bw
                    plsc.addupdate_scatter(y1d, [lin], val)
            return c2

        lax.fori_loop(0, EB // 16, grp, 0)
        return carry

    lax.fori_loop(0, nb, blk, 0)

    pltpu.sync_copy(y1d, h_out.at[pl.ds(f0, ROWS * F)])


def _step_call(h_in, sxc, srcb, dstb, wb, bnd):
    mesh = plsc.VectorSubcoreMesh(core_axis_name="c", subcore_axis_name="s")
    f = pl.kernel(
        _step_body,
        out_type=pltpu.HBM((NPAD * F,), jnp.float32),
        mesh=mesh,
        compiler_params=pltpu.CompilerParams(needs_layout_passes=False),
        scratch_types=[
            pltpu.VMEM((ROWS * F,), jnp.float32),
            pltpu.VMEM((EB, F), jnp.float32),
            pltpu.VMEM((EB,), jnp.int32),
            pltpu.VMEM((EB,), jnp.int32),
            pltpu.VMEM((EB,), jnp.float32),
            pltpu.VMEM((4, 16), jnp.int32),
        ],
    )
    return f(h_in, sxc, srcb, dstb, wb, bnd)


def kernel(x, edge_index, edge_weight, W1, b1, W2, b2, Wo, bo):
    src = edge_index[1]
    dst = edge_index[0]

    order = jnp.argsort(dst)
    dsts = dst[order]
    srcs = src[order]
    ws = edge_weight[order]

    starts = jnp.searchsorted(dsts, jnp.arange(NW, dtype=jnp.int32) * ROWS)
    starts = starts.astype(jnp.int32)
    ends = jnp.concatenate([starts[1:], jnp.array([E], jnp.int32)])
    b0 = (starts // EB) * EB
    nb = jnp.maximum(0, -((b0 - ends) // EB))
    bnd = jnp.stack([starts, ends, b0, nb])
    bnd = jnp.broadcast_to(bnd.T[:, :, None], (NW, 4, 16))
    bnd = bnd.reshape(NC, NS, 4, 16).astype(jnp.int32)

    pad = EPADL - E
    srcs = jnp.pad(srcs, (0, pad))
    dsts = jnp.pad(dsts, (0, pad))
    ws = jnp.pad(ws, (0, pad))

    xc, sxc, wsc = _prep_call(x, ws)
    sxc_f = sxc.reshape(-1)
    h = xc
    for _ in range(N_POWERS):
        h = _step_call(h, sxc_f, srcs, dsts, wsc, bnd).reshape(NPAD, F)
    xc2, sxc2 = _mid_call(h, W1, b1)
    sxc2_f = sxc2.reshape(-1)
    h = xc2
    for _ in range(N_POWERS):
        h = _step_call(h, sxc2_f, srcs, dsts, wsc, bnd).reshape(NPAD, F)
    return _out_call(h, W2, b2, Wo, jnp.reshape(bo, (1, C)))


# speedup vs baseline: 2.6584x; 1.1221x over previous
import functools

import jax
import jax.numpy as jnp
from jax import lax
from jax.experimental import pallas as pl
from jax.experimental.pallas import tpu as pltpu
from jax.experimental.pallas import tpu_sc as plsc

N = 10000
F = 128
E = 320000
C = 40
ALPHA = 1.0
N_POWERS = 10
C1 = ALPHA / (1.0 + ALPHA)
C2 = 1.0 / (1.0 + ALPHA)

NS = 16
NC = 2
NW = NC * NS
EB = 256
NPAD = 10240
ROWS = NPAD // NW
EPADL = E + 2 * EB


def _pad_rows(a):
    return jnp.concatenate(
        [a, jnp.zeros((NPAD - N, a.shape[1]), jnp.float32)], axis=0
    )


def _prep_body(x_ref, w_ref, xc_ref, sxc_ref, ws_ref):
    x = x_ref[...]
    m = jnp.mean(x, axis=0, keepdims=True)
    xc = x - m
    xc_ref[...] = _pad_rows(xc)
    sxc_ref[...] = _pad_rows(C2 * xc)
    ws_ref[...] = C1 * w_ref[...]


def _prep_call(x, w_sorted):
    return pl.pallas_call(
        _prep_body,
        out_shape=[
            jax.ShapeDtypeStruct((NPAD, F), jnp.float32),
            jax.ShapeDtypeStruct((NPAD, F), jnp.float32),
            jax.ShapeDtypeStruct(w_sorted.shape, jnp.float32),
        ],
    )(x, w_sorted)


def _mid_body(inv_ref, W_ref, b_ref, xc_ref, sxc_ref):
    h = jnp.dot(inv_ref[:N, :], W_ref[...], preferred_element_type=jnp.float32)
    h = h + b_ref[...]
    m = jnp.mean(h, axis=0, keepdims=True)
    xc = h - m
    xc_ref[...] = _pad_rows(xc)
    sxc_ref[...] = _pad_rows(C2 * xc)


def _mid_call(inv, W, b):
    return pl.pallas_call(
        _mid_body,
        out_shape=[
            jax.ShapeDtypeStruct((NPAD, F), jnp.float32),
            jax.ShapeDtypeStruct((NPAD, F), jnp.float32),
        ],
    )(inv, W, b)


def _out_body(inv_ref, W2_ref, b2_ref, Wo_ref, bo_ref, o_ref):
    h = jnp.dot(inv_ref[:N, :], W2_ref[...], preferred_element_type=jnp.float32)
    h = h + b2_ref[...]
    o_ref[...] = jnp.dot(h, Wo_ref[...], preferred_element_type=jnp.float32) + bo_ref[...]


def _out_call(inv, W2, b2, Wo, bo):
    return pl.pallas_call(
        _out_body,
        out_shape=jax.ShapeDtypeStruct((N, C), jnp.float32),
    )(inv, W2, b2, Wo, bo)


_BC_F = lax.GatherDimensionNumbers(
    offset_dims=(), collapsed_slice_dims=(0,), start_index_map=(0,)
)


def _bcast_lane(v16, j):
    idx = jnp.full((16, 1), j, dtype=jnp.int32)
    return lax.gather(
        v16, idx, _BC_F, (1,), mode=lax.GatherScatterMode.PROMISE_IN_BOUNDS
    )


def _step_body(h_in, sxc, srcb, dstb, wb, bnd, h_out,
               y1d, gbuf, src_v, dst_v, w_v, bvm):
    cid = lax.axis_index("c")
    sid = lax.axis_index("s")
    wid = cid * NS + sid
    f0 = pl.multiple_of(wid * (ROWS * F), ROWS * F)

    pltpu.sync_copy(sxc.at[pl.ds(f0, ROWS * F)], y1d)

    pltpu.sync_copy(bnd.at[cid, sid], bvm)
    s_ex = bvm[0]
    e_ex = bvm[1]
    b0 = bvm[2][0]
    nb = bvm[3][0]

    iota = lax.iota(jnp.int32, 16)
    rbase = wid * ROWS

    def blk(b, carry):
        e0 = pl.multiple_of(b0 + b * EB, EB)
        pltpu.sync_copy(srcb.at[pl.ds(e0, EB)], src_v)
        pltpu.sync_copy(dstb.at[pl.ds(e0, EB)], dst_v)
        pltpu.sync_copy(wb.at[pl.ds(e0, EB)], w_v)
        pltpu.sync_copy(h_in.at[src_v], gbuf)

        def grp(g, c2):
            gid = jnp.full((16,), e0 + g * 16, jnp.int32) + iota
            valid = (gid >= s_ex) & (gid < e_ex)
            w16 = jnp.where(valid, w_v[pl.ds(g * 16, 16)], 0.0)
            d16 = jnp.clip(dst_v[pl.ds(g * 16, 16)] - rbase, 0, ROWS - 1)
            for j in range(16):
                bw = _bcast_lane(w16, j)
                bd = _bcast_lane(d16, j) * F
                e = g * 16 + j
                for k in range(F // 16):
                    lin = bd + (iota + k * 16)
                    val = gbuf[e, pl.ds(k * 16, 16)] * bw
                    plsc.addupdate_scatter(y1d, [lin], val)
            return c2

        lax.fori_loop(0, EB // 16, grp, 0)
        return carry

    lax.fori_loop(0, nb, blk, 0)

    pltpu.sync_copy(y1d, h_out.at[pl.ds(f0, ROWS * F)])


def _step_call(h_in, sxc, srcb, dstb, wb, bnd):
    mesh = plsc.VectorSubcoreMesh(core_axis_name="c", subcore_axis_name="s")
    f = pl.kernel(
        _step_body,
        out_type=pltpu.HBM((NPAD * F,), jnp.float32),
        mesh=mesh,
        compiler_params=pltpu.CompilerParams(needs_layout_passes=False),
        scratch_types=[
            pltpu.VMEM((ROWS * F,), jnp.float32),
            pltpu.VMEM((EB, F), jnp.float32),
            pltpu.VMEM((EB,), jnp.int32),
            pltpu.VMEM((EB,), jnp.int32),
            pltpu.VMEM((EB,), jnp.float32),
            pltpu.VMEM((4, 16), jnp.int32),
        ],
    )
    return f(h_in, sxc, srcb, dstb, wb, bnd)


def kernel(x, edge_index, edge_weight, W1, b1, W2, b2, Wo, bo):
    src = edge_index[1]
    dst = edge_index[0]

    order = jnp.argsort(dst)
    dsts = dst[order]
    srcs = src[order]
    ws = edge_weight[order]

    starts = jnp.searchsorted(dsts, jnp.arange(NW, dtype=jnp.int32) * ROWS)
    starts = starts.astype(jnp.int32)
    ends = jnp.concatenate([starts[1:], jnp.array([E], jnp.int32)])
    b0 = (starts // EB) * EB
    nb = jnp.maximum(0, -((b0 - ends) // EB))
    bnd = jnp.stack([starts, ends, b0, nb])
    bnd = jnp.broadcast_to(bnd.T[:, :, None], (NW, 4, 16))
    bnd = bnd.reshape(NC, NS, 4, 16).astype(jnp.int32)

    pad = EPADL - E
    srcs = jnp.pad(srcs, (0, pad))
    dsts = jnp.pad(dsts, (0, pad))
    ws = jnp.pad(ws, (0, pad))

    xc, sxc, wsc = _prep_call(x, ws)
    sxc_f = sxc.reshape(-1)
    h = xc
    for _ in range(N_POWERS):
        h = _step_call(h, sxc_f, srcs, dsts, wsc, bnd).reshape(NPAD, F)
    xc2, sxc2 = _mid_call(h, W1, b1)
    sxc2_f = sxc2.reshape(-1)
    h = xc2
    for _ in range(N_POWERS):
        h = _step_call(h, sxc2_f, srcs, dsts, wsc, bnd).reshape(NPAD, F)
    return _out_call(h, W2, b2, Wo, jnp.reshape(bo, (1, C)))


# speedup vs baseline: 2.7740x; 1.0435x over previous
import functools

import jax
import jax.numpy as jnp
from jax import lax
from jax.experimental import pallas as pl
from jax.experimental.pallas import tpu as pltpu
from jax.experimental.pallas import tpu_sc as plsc

N = 10000
F = 128
E = 320000
C = 40
ALPHA = 1.0
N_POWERS = 10
C1 = ALPHA / (1.0 + ALPHA)
C2 = 1.0 / (1.0 + ALPHA)

NS = 16
NC = 2
NW = NC * NS
EB = 512
NPAD = 10240
ROWS = NPAD // NW
EPADL = E + 2 * EB


def _pad_rows(a):
    return jnp.concatenate(
        [a, jnp.zeros((NPAD - N, a.shape[1]), jnp.float32)], axis=0
    )


def _prep_body(x_ref, w_ref, xc_ref, sxc_ref, ws_ref):
    x = x_ref[...]
    m = jnp.mean(x, axis=0, keepdims=True)
    xc = x - m
    xc_ref[...] = _pad_rows(xc)
    sxc_ref[...] = _pad_rows(C2 * xc)
    ws_ref[...] = C1 * w_ref[...]


def _prep_call(x, w_sorted):
    return pl.pallas_call(
        _prep_body,
        out_shape=[
            jax.ShapeDtypeStruct((NPAD, F), jnp.float32),
            jax.ShapeDtypeStruct((NPAD, F), jnp.float32),
            jax.ShapeDtypeStruct(w_sorted.shape, jnp.float32),
        ],
    )(x, w_sorted)


def _mid_body(inv_ref, W_ref, b_ref, xc_ref, sxc_ref):
    h = jnp.dot(inv_ref[:N, :], W_ref[...], preferred_element_type=jnp.float32)
    h = h + b_ref[...]
    m = jnp.mean(h, axis=0, keepdims=True)
    xc = h - m
    xc_ref[...] = _pad_rows(xc)
    sxc_ref[...] = _pad_rows(C2 * xc)


def _mid_call(inv, W, b):
    return pl.pallas_call(
        _mid_body,
        out_shape=[
            jax.ShapeDtypeStruct((NPAD, F), jnp.float32),
            jax.ShapeDtypeStruct((NPAD, F), jnp.float32),
        ],
    )(inv, W, b)


def _out_body(inv_ref, W2_ref, b2_ref, Wo_ref, bo_ref, o_ref):
    h = jnp.dot(inv_ref[:N, :], W2_ref[...], preferred_element_type=jnp.float32)
    h = h + b2_ref[...]
    o_ref[...] = jnp.dot(h, Wo_ref[...], preferred_element_type=jnp.float32) + bo_ref[...]


def _out_call(inv, W2, b2, Wo, bo):
    return pl.pallas_call(
        _out_body,
        out_shape=jax.ShapeDtypeStruct((N, C), jnp.float32),
    )(inv, W2, b2, Wo, bo)


_BC_F = lax.GatherDimensionNumbers(
    offset_dims=(), collapsed_slice_dims=(0,), start_index_map=(0,)
)


def _bcast_lane(v16, j):
    idx = jnp.full((16, 1), j, dtype=jnp.int32)
    return lax.gather(
        v16, idx, _BC_F, (1,), mode=lax.GatherScatterMode.PROMISE_IN_BOUNDS
    )


def _step_body(h_in, sxc, srcb, dstb, wb, bnd, h_out,
               y1d, gbuf, src_v, dst_v, w_v, bvm):
    cid = lax.axis_index("c")
    sid = lax.axis_index("s")
    wid = cid * NS + sid
    f0 = pl.multiple_of(wid * (ROWS * F), ROWS * F)

    pltpu.sync_copy(sxc.at[pl.ds(f0, ROWS * F)], y1d)

    pltpu.sync_copy(bnd.at[cid, sid], bvm)
    s_ex = bvm[0]
    e_ex = bvm[1]
    b0 = bvm[2][0]
    nb = bvm[3][0]

    iota = lax.iota(jnp.int32, 16)
    rbase = wid * ROWS

    def blk(b, carry):
        e0 = pl.multiple_of(b0 + b * EB, EB)
        pltpu.sync_copy(srcb.at[pl.ds(e0, EB)], src_v)
        pltpu.sync_copy(dstb.at[pl.ds(e0, EB)], dst_v)
        pltpu.sync_copy(wb.at[pl.ds(e0, EB)], w_v)
        pltpu.sync_copy(h_in.at[src_v], gbuf)

        def grp(g, c2):
            gid = jnp.full((16,), e0 + g * 16, jnp.int32) + iota
            valid = (gid >= s_ex) & (gid < e_ex)
            w16 = jnp.where(valid, w_v[pl.ds(g * 16, 16)], 0.0)
            d16 = jnp.clip(dst_v[pl.ds(g * 16, 16)] - rbase, 0, ROWS - 1)
            for j in range(16):
                bw = _bcast_lane(w16, j)
                bd = _bcast_lane(d16, j) * F
                e = g * 16 + j
                for k in range(F // 16):
                    lin = bd + (iota + k * 16)
                    val = gbuf[e, pl.ds(k * 16, 16)] * bw
                    plsc.addupdate_scatter(y1d, [lin], val)
            return c2

        lax.fori_loop(0, EB // 16, grp, 0)
        return carry

    lax.fori_loop(0, nb, blk, 0)

    pltpu.sync_copy(y1d, h_out.at[pl.ds(f0, ROWS * F)])


def _step_call(h_in, sxc, srcb, dstb, wb, bnd):
    mesh = plsc.VectorSubcoreMesh(core_axis_name="c", subcore_axis_name="s")
    f = pl.kernel(
        _step_body,
        out_type=pltpu.HBM((NPAD * F,), jnp.float32),
        mesh=mesh,
        compiler_params=pltpu.CompilerParams(needs_layout_passes=False),
        scratch_types=[
            pltpu.VMEM((ROWS * F,), jnp.float32),
            pltpu.VMEM((EB, F), jnp.float32),
            pltpu.VMEM((EB,), jnp.int32),
            pltpu.VMEM((EB,), jnp.int32),
            pltpu.VMEM((EB,), jnp.float32),
            pltpu.VMEM((4, 16), jnp.int32),
        ],
    )
    return f(h_in, sxc, srcb, dstb, wb, bnd)


def kernel(x, edge_index, edge_weight, W1, b1, W2, b2, Wo, bo):
    src = edge_index[1]
    dst = edge_index[0]

    order = jnp.argsort(dst)
    dsts = dst[order]
    srcs = src[order]
    ws = edge_weight[order]

    starts = jnp.searchsorted(dsts, jnp.arange(NW, dtype=jnp.int32) * ROWS)
    starts = starts.astype(jnp.int32)
    ends = jnp.concatenate([starts[1:], jnp.array([E], jnp.int32)])
    b0 = (starts // EB) * EB
    nb = jnp.maximum(0, -((b0 - ends) // EB))
    bnd = jnp.stack([starts, ends, b0, nb])
    bnd = jnp.broadcast_to(bnd.T[:, :, None], (NW, 4, 16))
    bnd = bnd.reshape(NC, NS, 4, 16).astype(jnp.int32)

    pad = EPADL - E
    srcs = jnp.pad(srcs, (0, pad))
    dsts = jnp.pad(dsts, (0, pad))
    ws = jnp.pad(ws, (0, pad))

    xc, sxc, wsc = _prep_call(x, ws)
    sxc_f = sxc.reshape(-1)
    h = xc
    for _ in range(N_POWERS):
        h = _step_call(h, sxc_f, srcs, dsts, wsc, bnd).reshape(NPAD, F)
    xc2, sxc2 = _mid_call(h, W1, b1)
    sxc2_f = sxc2.reshape(-1)
    h = xc2
    for _ in range(N_POWERS):
        h = _step_call(h, sxc2_f, srcs, dsts, wsc, bnd).reshape(NPAD, F)
    return _out_call(h, W2, b2, Wo, jnp.reshape(bo, (1, C)))


# speedup vs baseline: 5.3782x; 1.9388x over previous
import functools

import jax
import jax.numpy as jnp
from jax import lax
from jax.experimental import pallas as pl
from jax.experimental.pallas import tpu as pltpu
from jax.experimental.pallas import tpu_sc as plsc

N = 10000
F = 128
E = 320000
C = 40
ALPHA = 1.0
N_POWERS = 10
C1 = ALPHA / (1.0 + ALPHA)
C2 = 1.0 / (1.0 + ALPHA)

NS = 16
NC = 2
NW = NC * NS
EB = 256
NPAD = 10240
ROWS = NPAD // NW
EPADL = E + 2 * EB


def _pad_rows(a):
    return jnp.concatenate(
        [a, jnp.zeros((NPAD - N, a.shape[1]), jnp.float32)], axis=0
    )


def _prep_body(x_ref, w_ref, xc_ref, sxc_ref, ws_ref):
    x = x_ref[...]
    m = jnp.mean(x, axis=0, keepdims=True)
    xc = x - m
    xc_ref[...] = _pad_rows(xc)
    sxc_ref[...] = _pad_rows(C2 * xc)
    ws_ref[...] = C1 * w_ref[...]


def _prep_call(x, w_sorted):
    return pl.pallas_call(
        _prep_body,
        out_shape=[
            jax.ShapeDtypeStruct((NPAD, F), jnp.float32),
            jax.ShapeDtypeStruct((NPAD, F), jnp.float32),
            jax.ShapeDtypeStruct(w_sorted.shape, jnp.float32),
        ],
    )(x, w_sorted)


def _mid_body(inv_ref, W_ref, b_ref, xc_ref, sxc_ref):
    h = jnp.dot(inv_ref[:N, :], W_ref[...], preferred_element_type=jnp.float32)
    h = h + b_ref[...]
    m = jnp.mean(h, axis=0, keepdims=True)
    xc = h - m
    xc_ref[...] = _pad_rows(xc)
    sxc_ref[...] = _pad_rows(C2 * xc)


def _mid_call(inv, W, b):
    return pl.pallas_call(
        _mid_body,
        out_shape=[
            jax.ShapeDtypeStruct((NPAD, F), jnp.float32),
            jax.ShapeDtypeStruct((NPAD, F), jnp.float32),
        ],
    )(inv, W, b)


def _out_body(inv_ref, W2_ref, b2_ref, Wo_ref, bo_ref, o_ref):
    h = jnp.dot(inv_ref[:N, :], W2_ref[...], preferred_element_type=jnp.float32)
    h = h + b2_ref[...]
    o_ref[...] = jnp.dot(h, Wo_ref[...], preferred_element_type=jnp.float32) + bo_ref[...]


def _out_call(inv, W2, b2, Wo, bo):
    return pl.pallas_call(
        _out_body,
        out_shape=jax.ShapeDtypeStruct((N, C), jnp.float32),
    )(inv, W2, b2, Wo, bo)


_BC_F = lax.GatherDimensionNumbers(
    offset_dims=(), collapsed_slice_dims=(0,), start_index_map=(0,)
)


def _bcast_lane(v16, j):
    idx = jnp.full((16, 1), j, dtype=jnp.int32)
    return lax.gather(
        v16, idx, _BC_F, (1,), mode=lax.GatherScatterMode.PROMISE_IN_BOUNDS
    )


def _step_body(h_in, sxc, srcb, dstb, wb, bnd, h_out,
               ysh, gbuf, msg, dloc, src_v, dst_v, w_v, bvm):
    cid = lax.axis_index("c")
    sid = lax.axis_index("s")
    wid = cid * NS + sid
    r0 = pl.multiple_of(wid * ROWS, ROWS)
    q0 = pl.multiple_of(sid * ROWS, ROWS)

    pltpu.sync_copy(sxc.at[pl.ds(r0, ROWS)], ysh.at[pl.ds(q0, ROWS)])

    pltpu.sync_copy(bnd.at[cid, sid], bvm)
    s_ex = bvm[0]
    e_ex = bvm[1]
    b0 = bvm[2][0]
    nb = bvm[3][0]

    iota = lax.iota(jnp.int32, 16)
    rbase = wid * ROWS

    def blk(b, carry):
        e0 = pl.multiple_of(b0 + b * EB, EB)
        pltpu.sync_copy(srcb.at[pl.ds(e0, EB)], src_v)
        pltpu.sync_copy(dstb.at[pl.ds(e0, EB)], dst_v)
        pltpu.sync_copy(wb.at[pl.ds(e0, EB)], w_v)
        pltpu.sync_copy(h_in.at[src_v], gbuf)

        def grp(g, c2):
            gid = jnp.full((16,), e0 + g * 16, jnp.int32) + iota
            valid = (gid >= s_ex) & (gid < e_ex)
            w16 = jnp.where(valid, w_v[pl.ds(g * 16, 16)], 0.0)
            d16 = jnp.clip(dst_v[pl.ds(g * 16, 16)] - cid * (NS * ROWS),
                           sid * ROWS, (sid + 1) * ROWS - 1)
            dloc[pl.ds(g * 16, 16)] = d16
            for j in range(16):
                bw = _bcast_lane(w16, j)
                e = g * 16 + j
                for k in range(F // 16):
                    msg[e, pl.ds(k * 16, 16)] = gbuf[e, pl.ds(k * 16, 16)] * bw
            return c2

        lax.fori_loop(0, EB // 16, grp, 0)
        pltpu.sync_copy(msg, ysh.at[dloc], add=True)
        return carry

    lax.fori_loop(0, nb, blk, 0)

    pltpu.sync_copy(ysh.at[pl.ds(q0, ROWS)], h_out.at[pl.ds(r0, ROWS)])


def _step_call(h_in, sxc, srcb, dstb, wb, bnd):
    mesh = plsc.VectorSubcoreMesh(core_axis_name="c", subcore_axis_name="s")
    f = pl.kernel(
        _step_body,
        out_type=pltpu.HBM((NPAD, F), jnp.float32),
        mesh=mesh,
        compiler_params=pltpu.CompilerParams(needs_layout_passes=False),
        scratch_types=[
            pltpu.VMEM_SHARED((NS * ROWS, F), jnp.float32),
            pltpu.VMEM((EB, F), jnp.float32),
            pltpu.VMEM((EB, F), jnp.float32),
            pltpu.VMEM((EB,), jnp.int32),
            pltpu.VMEM((EB,), jnp.int32),
            pltpu.VMEM((EB,), jnp.int32),
            pltpu.VMEM((EB,), jnp.float32),
            pltpu.VMEM((4, 16), jnp.int32),
        ],
    )
    return f(h_in, sxc, srcb, dstb, wb, bnd)


def kernel(x, edge_index, edge_weight, W1, b1, W2, b2, Wo, bo):
    src = edge_index[1]
    dst = edge_index[0]

    order = jnp.argsort(dst)
    dsts = dst[order]
    srcs = src[order]
    ws = edge_weight[order]

    starts = jnp.searchsorted(dsts, jnp.arange(NW, dtype=jnp.int32) * ROWS)
    starts = starts.astype(jnp.int32)
    ends = jnp.concatenate([starts[1:], jnp.array([E], jnp.int32)])
    b0 = (starts // EB) * EB
    nb = jnp.maximum(0, -((b0 - ends) // EB))
    bnd = jnp.stack([starts, ends, b0, nb])
    bnd = jnp.broadcast_to(bnd.T[:, :, None], (NW, 4, 16))
    bnd = bnd.reshape(NC, NS, 4, 16).astype(jnp.int32)

    pad = EPADL - E
    srcs = jnp.pad(srcs, (0, pad))
    dsts = jnp.pad(dsts, (0, pad))
    ws = jnp.pad(ws, (0, pad))

    xc, sxc, wsc = _prep_call(x, ws)
    h = xc
    for _ in range(N_POWERS):
        h = _step_call(h, sxc, srcs, dsts, wsc, bnd)
    xc2, sxc2 = _mid_call(h, W1, b1)
    h = xc2
    for _ in range(N_POWERS):
        h = _step_call(h, sxc2, srcs, dsts, wsc, bnd)
    return _out_call(h, W2, b2, Wo, jnp.reshape(bo, (1, C)))


# speedup vs baseline: 5.5684x; 1.0354x over previous
import functools

import jax
import jax.numpy as jnp
from jax import lax
from jax.experimental import pallas as pl
from jax.experimental.pallas import tpu as pltpu
from jax.experimental.pallas import tpu_sc as plsc

N = 10000
F = 128
E = 320000
C = 40
ALPHA = 1.0
N_POWERS = 10
C1 = ALPHA / (1.0 + ALPHA)
C2 = 1.0 / (1.0 + ALPHA)

NS = 16
NC = 2
NW = NC * NS
EB = 128
NPAD = 10240
ROWS = NPAD // NW
EPADL = E + 4 * EB


def _pad_rows(a):
    return jnp.concatenate(
        [a, jnp.zeros((NPAD - N, a.shape[1]), jnp.float32)], axis=0
    )


def _prep_body(x_ref, w_ref, xc_ref, sxc_ref, ws_ref):
    x = x_ref[...]
    m = jnp.mean(x, axis=0, keepdims=True)
    xc = x - m
    xc_ref[...] = _pad_rows(xc)
    sxc_ref[...] = _pad_rows(C2 * xc)
    ws_ref[...] = C1 * w_ref[...]


def _prep_call(x, w_sorted):
    return pl.pallas_call(
        _prep_body,
        out_shape=[
            jax.ShapeDtypeStruct((NPAD, F), jnp.float32),
            jax.ShapeDtypeStruct((NPAD, F), jnp.float32),
            jax.ShapeDtypeStruct(w_sorted.shape, jnp.float32),
        ],
    )(x, w_sorted)


def _mid_body(inv_ref, W_ref, b_ref, xc_ref, sxc_ref):
    h = jnp.dot(inv_ref[:N, :], W_ref[...], preferred_element_type=jnp.float32)
    h = h + b_ref[...]
    m = jnp.mean(h, axis=0, keepdims=True)
    xc = h - m
    xc_ref[...] = _pad_rows(xc)
    sxc_ref[...] = _pad_rows(C2 * xc)


def _mid_call(inv, W, b):
    return pl.pallas_call(
        _mid_body,
        out_shape=[
            jax.ShapeDtypeStruct((NPAD, F), jnp.float32),
            jax.ShapeDtypeStruct((NPAD, F), jnp.float32),
        ],
    )(inv, W, b)


def _out_body(inv_ref, W2_ref, b2_ref, Wo_ref, bo_ref, o_ref):
    h = jnp.dot(inv_ref[:N, :], W2_ref[...], preferred_element_type=jnp.float32)
    h = h + b2_ref[...]
    o_ref[...] = jnp.dot(h, Wo_ref[...], preferred_element_type=jnp.float32) + bo_ref[...]


def _out_call(inv, W2, b2, Wo, bo):
    return pl.pallas_call(
        _out_body,
        out_shape=jax.ShapeDtypeStruct((N, C), jnp.float32),
    )(inv, W2, b2, Wo, bo)


_BC_F = lax.GatherDimensionNumbers(
    offset_dims=(), collapsed_slice_dims=(0,), start_index_map=(0,)
)


def _bcast_lane(v16, j):
    idx = jnp.full((16, 1), j, dtype=jnp.int32)
    return lax.gather(
        v16, idx, _BC_F, (1,), mode=lax.GatherScatterMode.PROMISE_IN_BOUNDS
    )


def _step_body(h_in, sxc, srcb, dstb, wb, bnd, h_out,
               ysh, gbuf, msg, dloc, src_v, dst_v, w_v, bvm, gsem, asem):
    cid = lax.axis_index("c")
    sid = lax.axis_index("s")
    wid = cid * NS + sid
    r0 = pl.multiple_of(wid * ROWS, ROWS)
    q0 = pl.multiple_of(sid * ROWS, ROWS)

    pltpu.sync_copy(sxc.at[pl.ds(r0, ROWS)], ysh.at[pl.ds(q0, ROWS)])

    pltpu.sync_copy(bnd.at[cid, sid], bvm)
    s_ex = bvm[0]
    e_ex = bvm[1]
    b0 = pl.multiple_of(bvm[2][0], EB)
    nb = bvm[3][0]
    nbp = (nb + 1) // 2

    iota = lax.iota(jnp.int32, 16)
    rbase = wid * ROWS
    zero16 = jnp.zeros((16,), jnp.float32)

    def start_gather(i, e0):
        pltpu.sync_copy(srcb.at[pl.ds(e0, EB)], src_v.at[i])
        pltpu.sync_copy(dstb.at[pl.ds(e0, EB)], dst_v.at[i])
        pltpu.sync_copy(wb.at[pl.ds(e0, EB)], w_v.at[i])
        pltpu.async_copy(h_in.at[src_v.at[i]], gbuf.at[i], gsem.at[i])

    def wait_gather(i):
        pltpu.make_async_copy(h_in.at[src_v.at[i]], gbuf.at[i], gsem.at[i]).wait()

    def start_add(i):
        pltpu.async_copy(msg.at[i], ysh.at[dloc.at[i]], asem.at[i], add=True)

    def wait_add(i):
        pltpu.make_async_copy(msg.at[i], ysh.at[dloc.at[i]], asem.at[i]).wait()

    def compute(i, e0):
        def grp(g, c2):
            gid = jnp.full((16,), e0 + g * 16, jnp.int32) + iota
            valid = (gid >= s_ex) & (gid < e_ex)
            w16 = jnp.where(valid, w_v[i, pl.ds(g * 16, 16)], 0.0)
            d16 = jnp.clip(dst_v[i, pl.ds(g * 16, 16)] - cid * (NS * ROWS),
                           sid * ROWS, (sid + 1) * ROWS - 1)
            dloc[i, pl.ds(g * 16, 16)] = d16
            for j in range(16):
                bw = _bcast_lane(w16, j)
                e = g * 16 + j
                for k in range(F // 16):
                    msg[i, e, pl.ds(k * 16, 16)] = (
                        gbuf[i, e, pl.ds(k * 16, 16)] * bw
                    )
            return c2
        lax.fori_loop(0, EB // 16, grp, 0)

    start_gather(0, b0)
    for g in range(EB // 16):
        dloc[1, pl.ds(g * 16, 16)] = jnp.full((16,), rbase, jnp.int32)
        for e in range(16):
            for k in range(F // 16):
                msg[1, g * 16 + e, pl.ds(k * 16, 16)] = zero16
    start_add(1)

    def pair(p, carry):
        e0 = pl.multiple_of(b0 + (2 * p) * EB, EB)
        wait_gather(0)
        start_gather(1, e0 + EB)
        compute(0, e0)
        wait_add(1)
        start_add(0)
        wait_gather(1)
        start_gather(0, e0 + 2 * EB)
        compute(1, e0 + EB)
        wait_add(0)
        start_add(1)
        return carry

    lax.fori_loop(0, nbp, pair, 0)

    wait_add(1)
    wait_gather(0)

    pltpu.sync_copy(ysh.at[pl.ds(q0, ROWS)], h_out.at[pl.ds(r0, ROWS)])


def _step_call(h_in, sxc, srcb, dstb, wb, bnd):
    mesh = plsc.VectorSubcoreMesh(core_axis_name="c", subcore_axis_name="s")
    f = pl.kernel(
        _step_body,
        out_type=pltpu.HBM((NPAD, F), jnp.float32),
        mesh=mesh,
        compiler_params=pltpu.CompilerParams(needs_layout_passes=False),
        scratch_types=[
            pltpu.VMEM_SHARED((NS * ROWS, F), jnp.float32),
            pltpu.VMEM((2, EB, F), jnp.float32),
            pltpu.VMEM((2, EB, F), jnp.float32),
            pltpu.VMEM((2, EB), jnp.int32),
            pltpu.VMEM((2, EB), jnp.int32),
            pltpu.VMEM((2, EB), jnp.int32),
            pltpu.VMEM((2, EB), jnp.float32),
            pltpu.VMEM((4, 16), jnp.int32),
            pltpu.SemaphoreType.DMA((2,)),
            pltpu.SemaphoreType.DMA((2,)),
        ],
    )
    return f(h_in, sxc, srcb, dstb, wb, bnd)


def kernel(x, edge_index, edge_weight, W1, b1, W2, b2, Wo, bo):
    src = edge_index[1]
    dst = edge_index[0]

    order = jnp.argsort(dst)
    dsts = dst[order]
    srcs = src[order]
    ws = edge_weight[order]

    starts = jnp.searchsorted(dsts, jnp.arange(NW, dtype=jnp.int32) * ROWS)
    starts = starts.astype(jnp.int32)
    ends = jnp.concatenate([starts[1:], jnp.array([E], jnp.int32)])
    b0 = (starts // EB) * EB
    nb = jnp.maximum(0, -((b0 - ends) // EB))
    bnd = jnp.stack([starts, ends, b0, nb])
    bnd = jnp.broadcast_to(bnd.T[:, :, None], (NW, 4, 16))
    bnd = bnd.reshape(NC, NS, 4, 16).astype(jnp.int32)

    pad = EPADL - E
    srcs = jnp.pad(srcs, (0, pad))
    dsts = jnp.pad(dsts, (0, pad))
    ws = jnp.pad(ws, (0, pad))

    xc, sxc, wsc = _prep_call(x, ws)
    h = xc
    for _ in range(N_POWERS):
        h = _step_call(h, sxc, srcs, dsts, wsc, bnd)
    xc2, sxc2 = _mid_call(h, W1, b1)
    h = xc2
    for _ in range(N_POWERS):
        h = _step_call(h, sxc2, srcs, dsts, wsc, bnd)
    return _out_call(h, W2, b2, Wo, jnp.reshape(bo, (1, C)))
